# baseline (speedup 1.0000x reference)
import functools

import jax
import jax.numpy as jnp
from jax import lax
from jax.experimental import pallas as pl
from jax.experimental.pallas import tpu as pltpu
from jax.experimental.pallas import tpu_sc as plsc

H = 16
B = 16384
NC = 2
NS = 16
NW = NC * NS
BPW = B // NW
G = 16
NG = BPW // G
BLK = 8 * H
RBLK = 16 * H

_mesh = plsc.VectorSubcoreMesh(core_axis_name="c", subcore_axis_name="s")


@functools.partial(
    pl.kernel,
    out_type=jax.ShapeDtypeStruct((NW, H), jnp.float32),
    mesh=_mesh,
    scratch_types=[
        pltpu.VMEM((BPW,), jnp.int32),
        pltpu.VMEM((BPW,), jnp.int32),
        pltpu.VMEM((2, G, BLK), jnp.float32),
        pltpu.VMEM((2, G, BLK), jnp.float32),
        pltpu.VMEM((H,), jnp.float32),
        pltpu.SemaphoreType.DMA,
        pltpu.SemaphoreType.DMA,
    ],
    compiler_params=pltpu.CompilerParams(needs_layout_passes=False),
)
def _partials_sc(rid_hbm, pid_hbm, rlin_hbm, plin_hbm, out_hbm,
                 idx_r, idx_p, rblk, pblk, acc_v, semA, semB):
    wid = lax.axis_index("s") * NC + lax.axis_index("c")
    base = wid * BPW
    pltpu.sync_copy(rid_hbm.at[pl.ds(base, BPW)], idx_r)
    pltpu.sync_copy(pid_hbm.at[pl.ds(base, BPW)], idx_p)

    iota16 = lax.iota(jnp.int32, 16)
    sems = (semA, semB)

    def vecs(group):
        off = pl.multiple_of(group * G, G)
        return idx_r[pl.ds(off, G)], idx_p[pl.ds(off, G)]

    def enqueue(group, bset):
        sem = sems[bset]
        rvec, pvec = vecs(group)
        pq = lax.shift_right_logical(pvec, 3)
        pltpu.async_copy(plin_hbm.at[pq], pblk.at[bset], sem)
        for e in range(G):
            rq = lax.shift_right_logical(rvec[e], 3)
            pltpu.async_copy(rlin_hbm.at[rq], rblk.at[bset, e], sem)

    def process(group, bset, acc):
        sem = sems[bset]
        rvec, pvec = vecs(group)
        pq = lax.shift_right_logical(pvec, 3)
        pltpu.make_async_copy(plin_hbm.at[pq], pblk.at[bset], sem).wait()
        for e in range(G):
            pltpu.make_async_copy(rlin_hbm.at[0], rblk.at[bset, e], sem).wait()
        for e in range(G):
            rsub = jnp.bitwise_and(rvec[e], 7) * H
            rv = plsc.load_gather(rblk.at[bset, e], [iota16 + rsub])
            psub = jnp.bitwise_and(pvec[e], 7) * H
            pv = plsc.load_gather(pblk.at[bset, e], [iota16 + psub])
            acc = acc + rv * pv
        return acc

    enqueue(0, 0)
    enqueue(1, 1)

    def body(h, acc):
        g = h * 2
        acc = process(g, 0, acc)

        @pl.when(g + 2 < NG)
        def _():
            enqueue(g + 2, 0)

        acc = process(g + 1, 1, acc)

        @pl.when(g + 3 < NG)
        def _():
            enqueue(g + 3, 1)

        return acc

    acc = lax.fori_loop(0, NG // 2, body, jnp.zeros((H,), jnp.float32))

    acc_v[...] = acc
    pltpu.sync_copy(acc_v, out_hbm.at[wid])


def _sum_tc(p_ref, o_ref):
    o_ref[...] = jnp.sum(p_ref[...], axis=0, keepdims=True)


@jax.jit
def kernel(reviewer_id, product_id, reviewer_table, product_table):
    rlin = reviewer_table.reshape(reviewer_table.shape[0] // 8, BLK)
    plin = product_table.reshape(product_table.shape[0] // 8, BLK)
    partials = _partials_sc(reviewer_id, product_id, rlin, plin)
    out = pl.pallas_call(
        _sum_tc,
        out_shape=jax.ShapeDtypeStruct((1, H), jnp.float32),
    )(partials)
    return out.reshape(H)


# speedup vs baseline: 1.5265x; 1.5265x over previous
import functools

import jax
import jax.numpy as jnp
from jax import lax
from jax.experimental import pallas as pl
from jax.experimental.pallas import tpu as pltpu
from jax.experimental.pallas import tpu_sc as plsc

H = 16
B = 16384
NC = 2
NS = 16
NW = NC * NS
BPW = B // NW
G = 16
NG = BPW // G
BLK = 8 * H

_mesh = plsc.VectorSubcoreMesh(core_axis_name="c", subcore_axis_name="s")


@functools.partial(
    pl.kernel,
    out_type=jax.ShapeDtypeStruct((NW, H), jnp.float32),
    mesh=_mesh,
    scratch_types=[
        pltpu.VMEM((BPW,), jnp.int32),
        pltpu.VMEM((BPW,), jnp.int32),
        pltpu.VMEM((2, G, H), jnp.float32),
        pltpu.VMEM((2, G, BLK), jnp.float32),
        pltpu.VMEM((H,), jnp.float32),
        pltpu.SemaphoreType.DMA,
        pltpu.SemaphoreType.DMA,
    ],
    compiler_params=pltpu.CompilerParams(needs_layout_passes=False),
)
def _partials_sc(rid_hbm, pid_hbm, rlin_hbm, plin_hbm, out_hbm,
                 idx_r, idx_p, rblk, pblk, acc_v, semA, semB):
    wid = lax.axis_index("s") * NC + lax.axis_index("c")
    base = wid * BPW
    pltpu.sync_copy(rid_hbm.at[pl.ds(base, BPW)], idx_r)
    pltpu.sync_copy(pid_hbm.at[pl.ds(base, BPW)], idx_p)

    iota16 = lax.iota(jnp.int32, 16)
    sems = (semA, semB)

    def vecs(group):
        off = pl.multiple_of(group * G, G)
        return idx_r[pl.ds(off, G)], idx_p[pl.ds(off, G)]

    def enqueue(group, bset):
        sem = sems[bset]
        rvec, pvec = vecs(group)
        pq = lax.shift_right_logical(pvec, 3)
        pltpu.async_copy(plin_hbm.at[pq], pblk.at[bset], sem)
        for e in range(G):
            pltpu.async_copy(rlin_hbm.at[rvec[e]], rblk.at[bset, e], sem)

    def process(group, bset, acc):
        sem = sems[bset]
        rvec, pvec = vecs(group)
        pq = lax.shift_right_logical(pvec, 3)
        pltpu.make_async_copy(plin_hbm.at[pq], pblk.at[bset], sem).wait()
        for e in range(G):
            pltpu.make_async_copy(rlin_hbm.at[0], rblk.at[bset, e], sem).wait()
        for e in range(G):
            rv = rblk[bset, e]
            psub = jnp.bitwise_and(pvec[e], 7) * H
            pv = plsc.load_gather(pblk.at[bset, e], [iota16 + psub])
            acc = acc + rv * pv
        return acc

    enqueue(0, 0)
    enqueue(1, 1)

    def body(h, acc):
        g = h * 2
        acc = process(g, 0, acc)

        @pl.when(g + 2 < NG)
        def _():
            enqueue(g + 2, 0)

        acc = process(g + 1, 1, acc)

        @pl.when(g + 3 < NG)
        def _():
            enqueue(g + 3, 1)

        return acc

    acc = lax.fori_loop(0, NG // 2, body, jnp.zeros((H,), jnp.float32))

    acc_v[...] = acc
    pltpu.sync_copy(acc_v, out_hbm.at[wid])


def _sum_tc(p_ref, o_ref):
    o_ref[...] = jnp.sum(p_ref[...], axis=0, keepdims=True)


@jax.jit
def kernel(reviewer_id, product_id, reviewer_table, product_table):
    rlin = reviewer_table
    plin = product_table.reshape(product_table.shape[0] // 8, BLK)
    partials = _partials_sc(reviewer_id, product_id, rlin, plin)
    out = pl.pallas_call(
        _sum_tc,
        out_shape=jax.ShapeDtypeStruct((1, H), jnp.float32),
    )(partials)
    return out.reshape(H)
